# baseline (speedup 1.0000x reference)
import jax
import jax.numpy as jnp
from jax.experimental import pallas as pl
from jax.experimental.pallas import tpu as pltpu

_ROWS = 2048


def _copy_kernel(in_ref, out_ref):
    out_ref[...] = in_ref[...]


def kernel(x):
    n = x.size // 512
    xf = x.reshape(n, 512)
    out = pl.pallas_call(
        _copy_kernel,
        grid=(n // _ROWS,),
        in_specs=[pl.BlockSpec((_ROWS, 512), lambda i: (i, 0))],
        out_specs=pl.BlockSpec((_ROWS, 512), lambda i: (i, 0)),
        out_shape=jax.ShapeDtypeStruct((n, 512), x.dtype),
        compiler_params=pltpu.CompilerParams(
            dimension_semantics=("parallel",),
        ),
    )(xf)
    return out.reshape(x.shape)
